# baseline (speedup 1.0000x reference)
import math

import jax
import jax.numpy as jnp
from jax.experimental import pallas as pl
from jax.experimental.pallas import tpu as pltpu

_NEG = -1e30
_LARGE_NUMBER = 1e10
_K = 32
_S = 2


def _lae(a, b):
    return jnp.maximum(a, b) + jnp.log1p(jnp.exp(-jnp.abs(a - b)))


def _dp_body(theta_ref, u0_ref, u1_ref, marg_ref, m0_ref, m1_ref, btab_ref):
    N = theta_ref.shape[0]
    B = theta_ref.shape[-1]
    k = btab_ref.shape[1] - 1
    neg = jnp.float32(_NEG)

    binit = jnp.concatenate(
        [jnp.zeros((1, B), jnp.float32), jnp.full((k, B), neg, jnp.float32)],
        axis=0)
    btab_ref[N] = binit

    def mk_bwd(rows):
        def bwd(t, bcur):
            j = N - 1 - t
            th = theta_ref[j]
            shifted = jnp.concatenate(
                [jnp.full((1, B), neg, jnp.float32), bcur[:-1]], axis=0)
            bnew = _lae(bcur, th + shifted)
            btab_ref[j, :rows] = bnew
            if rows < k + 1:
                btab_ref[j, rows:] = jnp.full((k + 1 - rows, B), neg,
                                              jnp.float32)
            return bnew
        return bwd

    bcur = binit[:8]
    t_lo = 0
    for p in range(1, (k + 8) // 8 + 1):
        rows = min(8 * p, k + 1)
        t_hi = 8 * p - 1 if rows < k + 1 else N
        bcur = jax.lax.fori_loop(t_lo, t_hi, mk_bwd(rows), bcur,
                                 unroll=False)
        if rows < k + 1:
            bcur = jnp.concatenate(
                [bcur, jnp.full((min(8, k + 1 - rows), B), neg,
                                jnp.float32)], axis=0)
        t_lo = t_hi

    logz = btab_ref[0, k:k + 1]

    r_init = jnp.full((1, B), k, jnp.int32)

    def mk_fwd(a):
        def fwd(i, carry):
            g, r0, r1, den0, den1 = carry
            th = theta_ref[i]
            bip1 = btab_ref[i + 1]

            s = jnp.sum(jnp.exp(g + bip1[a:k]), axis=0, keepdims=True)
            marg = jnp.exp(th) * s
            marg_ref[i] = marg

            def gather33(idx):
                tsel = idx >> 3
                z = bip1[a:a + 8]
                for tb in range(a // 8 + 1, k // 8):
                    z = jnp.where(tsel >= tb, bip1[8 * tb:8 * tb + 8], z)
                gv = jnp.take_along_axis(z, idx & 7, axis=0)
                return jnp.where(idx >= k, bip1[k:k + 1], gv)

            def samp(r, den, u):
                num_g = gather33(jnp.maximum(r - 1, 0))
                p = jnp.exp(th + num_g - den)
                inc = (u < p) & (r > 0)
                den_next = jnp.where(inc, num_g, gather33(r))
                return (r - inc.astype(jnp.int32), den_next,
                        inc.astype(jnp.float32))

            r0n, den0n, inc0 = samp(r0, den0, u0_ref[i])
            r1n, den1n, inc1 = samp(r1, den1, u1_ref[i])
            m0_ref[i] = (inc0 - marg) + marg
            m1_ref[i] = (inc1 - marg) + marg

            gshift = jnp.concatenate(
                [g[1:], jnp.full((1, B), neg, jnp.float32)], axis=0)
            gnew = _lae(g, th + gshift)
            return gnew, r0n, r1n, den0n, den1n
        return fwd

    g = jnp.concatenate(
        [jnp.full((7, B), neg, jnp.float32), -logz], axis=0)
    carry = (g, r_init, r_init, logz, logz)
    i_lo = 0
    for a in range(k - 8, -1, -8):
        i_hi = k - 1 - a if a > 0 else N
        carry = jax.lax.fori_loop(i_lo, i_hi, mk_fwd(a), carry,
                                  unroll=False)
        if a > 0:
            carry = (jnp.concatenate(
                [jnp.full((8, B), neg, jnp.float32), carry[0]], axis=0),
            ) + carry[1:]
        i_lo = i_hi


def _run_blocks(theta_t, u0, u1, n_pow2, k, block_b, interpret=False):
    rp = theta_t.shape[-1]
    grid = rp // block_b
    spec = pl.BlockSpec((n_pow2, 1, block_b), lambda b: (0, 0, b))
    shape = jax.ShapeDtypeStruct((n_pow2, 1, rp), jnp.float32)
    return pl.pallas_call(
        _dp_body,
        grid=(grid,),
        in_specs=[spec, spec, spec],
        out_specs=[spec, spec, spec],
        out_shape=[shape, shape, shape],
        scratch_shapes=[pltpu.VMEM((n_pow2 + 1, k + 1, block_b), jnp.float32)],
        interpret=interpret,
    )(theta_t, u0, u1)


def kernel(scores):
    nnodes, choices, ensemble = scores.shape
    rows = nnodes * ensemble
    k = min(_K, choices)
    n_pow2 = 2 ** int(math.ceil(math.log2(choices)))

    theta_t = jnp.transpose(scores, (1, 0, 2)).reshape(choices, rows)
    if n_pow2 > choices:
        theta_t = jnp.concatenate(
            [theta_t,
         jnp.full((n_pow2 - choices, rows), -_LARGE_NUMBER, theta_t.dtype)],
            axis=0)
    u = jax.random.uniform(jax.random.key(1), (n_pow2, _S, rows),
                           dtype=theta_t.dtype)

    block_b = 1024
    rp = ((rows + block_b - 1) // block_b) * block_b
    pad = rp - rows
    theta_p = jnp.pad(theta_t, ((0, 0), (0, pad)))[:, None, :]
    u0 = jnp.pad(u[:, 0], ((0, 0), (0, pad)), constant_values=2.0)[:, None, :]
    u1 = jnp.pad(u[:, 1], ((0, 0), (0, pad)), constant_values=2.0)[:, None, :]

    marg_t, mk0, mk1 = _run_blocks(theta_p, u0, u1, n_pow2, k, block_b)

    marginals = (marg_t[:choices, 0, :rows]
                 .reshape(choices, nnodes, ensemble).transpose(1, 0, 2))
    sb = jnp.stack([mk0[:choices, 0, :rows], mk1[:choices, 0, :rows]])
    samples = (sb.transpose(0, 2, 1)
               .reshape(_S, nnodes, ensemble, choices).transpose(0, 1, 3, 2))
    return samples, marginals


# speedup vs baseline: 1.0080x; 1.0080x over previous
import math

import jax
import jax.numpy as jnp
from jax.experimental import pallas as pl
from jax.experimental.pallas import tpu as pltpu

_NEG = -1e30
_LARGE_NUMBER = 1e10
_K = 32
_S = 2


def _lae(a, b):
    return jnp.maximum(a, b) + jnp.log1p(jnp.exp(-jnp.abs(a - b)))


def _dp_body(theta_ref, u0_ref, u1_ref, marg_ref, m0_ref, m1_ref, btab_ref):
    N = theta_ref.shape[0]
    B = theta_ref.shape[-1]
    k = btab_ref.shape[1] - 1
    neg = jnp.float32(_NEG)

    binit = jnp.concatenate(
        [jnp.zeros((1, B), jnp.float32), jnp.full((k, B), neg, jnp.float32)],
        axis=0)
    btab_ref[N] = binit

    def mk_bwd(rows):
        def bwd(t, bcur):
            j = N - 1 - t
            th = theta_ref[j]
            shifted = jnp.concatenate(
                [jnp.full((1, B), neg, jnp.float32), bcur[:-1]], axis=0)
            bnew = _lae(bcur, th + shifted)
            btab_ref[j, :rows] = bnew
            if rows < k + 1:
                btab_ref[j, rows:] = jnp.full((k + 1 - rows, B), neg,
                                              jnp.float32)
            return bnew
        return bwd

    bcur = binit[:8]
    t_lo = 0
    for p in range(1, (k + 8) // 8 + 1):
        rows = min(8 * p, k + 1)
        t_hi = 8 * p - 1 if rows < k + 1 else N
        bcur = jax.lax.fori_loop(t_lo, t_hi, mk_bwd(rows), bcur,
                                 unroll=2 if rows == k + 1 else False)
        if rows < k + 1:
            bcur = jnp.concatenate(
                [bcur, jnp.full((min(8, k + 1 - rows), B), neg,
                                jnp.float32)], axis=0)
        t_lo = t_hi

    logz = btab_ref[0, k:k + 1]

    r_init = jnp.full((1, B), k, jnp.int32)

    def mk_fwd(a):
        def fwd(i, carry):
            g, r0, r1, den0, den1 = carry
            th = theta_ref[i]
            bip1 = btab_ref[i + 1]

            s = jnp.sum(jnp.exp(g + bip1[a:k]), axis=0, keepdims=True)
            marg = jnp.exp(th) * s
            marg_ref[i] = marg

            def gather33(idx):
                tsel = idx >> 3
                z = bip1[a:a + 8]
                for tb in range(a // 8 + 1, k // 8):
                    z = jnp.where(tsel >= tb, bip1[8 * tb:8 * tb + 8], z)
                gv = jnp.take_along_axis(z, idx & 7, axis=0)
                return jnp.where(idx >= k, bip1[k:k + 1], gv)

            def samp(r, den, u):
                num_g = gather33(jnp.maximum(r - 1, 0))
                p = jnp.exp(th + num_g - den)
                inc = (u < p) & (r > 0)
                den_next = jnp.where(inc, num_g, gather33(r))
                return (r - inc.astype(jnp.int32), den_next,
                        inc.astype(jnp.float32))

            r0n, den0n, inc0 = samp(r0, den0, u0_ref[i])
            r1n, den1n, inc1 = samp(r1, den1, u1_ref[i])
            m0_ref[i] = (inc0 - marg) + marg
            m1_ref[i] = (inc1 - marg) + marg

            gshift = jnp.concatenate(
                [g[1:], jnp.full((1, B), neg, jnp.float32)], axis=0)
            gnew = _lae(g, th + gshift)
            return gnew, r0n, r1n, den0n, den1n
        return fwd

    g = jnp.concatenate(
        [jnp.full((7, B), neg, jnp.float32), -logz], axis=0)
    carry = (g, r_init, r_init, logz, logz)
    i_lo = 0
    for a in range(k - 8, -1, -8):
        i_hi = k - 1 - a if a > 0 else N
        carry = jax.lax.fori_loop(i_lo, i_hi, mk_fwd(a), carry,
                                  unroll=2 if a == 0 else False)
        if a > 0:
            carry = (jnp.concatenate(
                [jnp.full((8, B), neg, jnp.float32), carry[0]], axis=0),
            ) + carry[1:]
        i_lo = i_hi


def _run_blocks(theta_t, u0, u1, n_pow2, k, block_b, interpret=False):
    rp = theta_t.shape[-1]
    grid = rp // block_b
    spec = pl.BlockSpec((n_pow2, 1, block_b), lambda b: (0, 0, b))
    shape = jax.ShapeDtypeStruct((n_pow2, 1, rp), jnp.float32)
    return pl.pallas_call(
        _dp_body,
        grid=(grid,),
        in_specs=[spec, spec, spec],
        out_specs=[spec, spec, spec],
        out_shape=[shape, shape, shape],
        scratch_shapes=[pltpu.VMEM((n_pow2 + 1, k + 1, block_b), jnp.float32)],
        interpret=interpret,
    )(theta_t, u0, u1)


def kernel(scores):
    nnodes, choices, ensemble = scores.shape
    rows = nnodes * ensemble
    k = min(_K, choices)
    n_pow2 = 2 ** int(math.ceil(math.log2(choices)))

    theta_t = jnp.transpose(scores, (1, 0, 2)).reshape(choices, rows)
    if n_pow2 > choices:
        theta_t = jnp.concatenate(
            [theta_t,
         jnp.full((n_pow2 - choices, rows), -_LARGE_NUMBER, theta_t.dtype)],
            axis=0)
    u = jax.random.uniform(jax.random.key(1), (n_pow2, _S, rows),
                           dtype=theta_t.dtype)

    block_b = 1024
    rp = ((rows + block_b - 1) // block_b) * block_b
    pad = rp - rows
    theta_p = jnp.pad(theta_t, ((0, 0), (0, pad)))[:, None, :]
    u0 = jnp.pad(u[:, 0], ((0, 0), (0, pad)), constant_values=2.0)[:, None, :]
    u1 = jnp.pad(u[:, 1], ((0, 0), (0, pad)), constant_values=2.0)[:, None, :]

    marg_t, mk0, mk1 = _run_blocks(theta_p, u0, u1, n_pow2, k, block_b)

    marginals = (marg_t[:choices, 0, :rows]
                 .reshape(choices, nnodes, ensemble).transpose(1, 0, 2))
    sb = jnp.stack([mk0[:choices, 0, :rows], mk1[:choices, 0, :rows]])
    samples = (sb.transpose(0, 2, 1)
               .reshape(_S, nnodes, ensemble, choices).transpose(0, 1, 3, 2))
    return samples, marginals


# speedup vs baseline: 1.0090x; 1.0010x over previous
import math

import jax
import jax.numpy as jnp
from jax.experimental import pallas as pl
from jax.experimental.pallas import tpu as pltpu

_NEG = -1e30
_LARGE_NUMBER = 1e10
_K = 32
_S = 2


def _lae(a, b):
    return jnp.maximum(a, b) + jnp.log1p(jnp.exp(-jnp.abs(a - b)))


def _dp_body(theta_ref, u0_ref, u1_ref, marg_ref, m0_ref, m1_ref, btab_ref):
    N = theta_ref.shape[0]
    B = theta_ref.shape[-1]
    k = btab_ref.shape[1] - 1
    neg = jnp.float32(_NEG)

    binit = jnp.concatenate(
        [jnp.zeros((1, B), jnp.float32), jnp.full((k, B), neg, jnp.float32)],
        axis=0)
    btab_ref[N] = binit

    def mk_bwd(rows):
        def bwd(t, bcur):
            j = N - 1 - t
            th = theta_ref[j]
            shifted = jnp.concatenate(
                [jnp.full((1, B), neg, jnp.float32), bcur[:-1]], axis=0)
            bnew = _lae(bcur, th + shifted)
            btab_ref[j, :rows] = bnew
            if rows < k + 1:
                btab_ref[j, rows:] = jnp.full((k + 1 - rows, B), neg,
                                              jnp.float32)
            return bnew
        return bwd

    bcur = binit[:8]
    t_lo = 0
    for p in range(1, (k + 8) // 8 + 1):
        rows = min(8 * p, k + 1)
        t_hi = 8 * p - 1 if rows < k + 1 else N
        bcur = jax.lax.fori_loop(t_lo, t_hi, mk_bwd(rows), bcur,
                                 unroll=2 if rows == k + 1 else False)
        if rows < k + 1:
            bcur = jnp.concatenate(
                [bcur, jnp.full((min(8, k + 1 - rows), B), neg,
                                jnp.float32)], axis=0)
        t_lo = t_hi

    logz = btab_ref[0, k:k + 1]

    r_init = jnp.full((1, B), k, jnp.int32)

    def mk_fwd(a):
        def fwd(i, carry):
            g, r0, r1, den0, den1 = carry
            th = theta_ref[i]
            bip1 = btab_ref[i + 1]

            s = jnp.sum(jnp.exp(g + bip1[a:k]), axis=0, keepdims=True)
            marg = jnp.exp(th) * s
            marg_ref[i] = marg

            def gather33(idx, can_hit_last):
                tsel = idx >> 3
                z = bip1[a:a + 8]
                for tb in range(a // 8 + 1, k // 8):
                    z = jnp.where(tsel >= tb, bip1[8 * tb:8 * tb + 8], z)
                gv = jnp.take_along_axis(z, idx & 7, axis=0)
                if can_hit_last:
                    gv = jnp.where(idx >= k, bip1[k:k + 1], gv)
                return gv

            def samp(r, den, u):
                num_g = gather33(jnp.maximum(r - 1, 0), False)
                p = jnp.exp(th + num_g - den)
                inc = (u < p) & (r > 0)
                den_next = jnp.where(inc, num_g, gather33(r, True))
                return (r - inc.astype(jnp.int32), den_next,
                        inc.astype(jnp.float32))

            r0n, den0n, inc0 = samp(r0, den0, u0_ref[i])
            r1n, den1n, inc1 = samp(r1, den1, u1_ref[i])
            m0_ref[i] = (inc0 - marg) + marg
            m1_ref[i] = (inc1 - marg) + marg

            gshift = jnp.concatenate(
                [g[1:], jnp.full((1, B), neg, jnp.float32)], axis=0)
            gnew = _lae(g, th + gshift)
            return gnew, r0n, r1n, den0n, den1n
        return fwd

    g = jnp.concatenate(
        [jnp.full((7, B), neg, jnp.float32), -logz], axis=0)
    carry = (g, r_init, r_init, logz, logz)
    i_lo = 0
    for a in range(k - 8, -1, -8):
        i_hi = k - 1 - a if a > 0 else N
        carry = jax.lax.fori_loop(i_lo, i_hi, mk_fwd(a), carry,
                                  unroll=2 if a == 0 else False)
        if a > 0:
            carry = (jnp.concatenate(
                [jnp.full((8, B), neg, jnp.float32), carry[0]], axis=0),
            ) + carry[1:]
        i_lo = i_hi


def _run_blocks(theta_t, u0, u1, n_pow2, k, block_b, interpret=False):
    rp = theta_t.shape[-1]
    grid = rp // block_b
    spec = pl.BlockSpec((n_pow2, 1, block_b), lambda b: (0, 0, b))
    shape = jax.ShapeDtypeStruct((n_pow2, 1, rp), jnp.float32)
    return pl.pallas_call(
        _dp_body,
        grid=(grid,),
        in_specs=[spec, spec, spec],
        out_specs=[spec, spec, spec],
        out_shape=[shape, shape, shape],
        scratch_shapes=[pltpu.VMEM((n_pow2 + 1, k + 1, block_b), jnp.float32)],
        interpret=interpret,
    )(theta_t, u0, u1)


def kernel(scores):
    nnodes, choices, ensemble = scores.shape
    rows = nnodes * ensemble
    k = min(_K, choices)
    n_pow2 = 2 ** int(math.ceil(math.log2(choices)))

    theta_t = jnp.transpose(scores, (1, 0, 2)).reshape(choices, rows)
    if n_pow2 > choices:
        theta_t = jnp.concatenate(
            [theta_t,
         jnp.full((n_pow2 - choices, rows), -_LARGE_NUMBER, theta_t.dtype)],
            axis=0)
    u = jax.random.uniform(jax.random.key(1), (n_pow2, _S, rows),
                           dtype=theta_t.dtype)

    block_b = 1024
    rp = ((rows + block_b - 1) // block_b) * block_b
    pad = rp - rows
    theta_p = jnp.pad(theta_t, ((0, 0), (0, pad)))[:, None, :]
    u0 = jnp.pad(u[:, 0], ((0, 0), (0, pad)), constant_values=2.0)[:, None, :]
    u1 = jnp.pad(u[:, 1], ((0, 0), (0, pad)), constant_values=2.0)[:, None, :]

    marg_t, mk0, mk1 = _run_blocks(theta_p, u0, u1, n_pow2, k, block_b)

    marginals = (marg_t[:choices, 0, :rows]
                 .reshape(choices, nnodes, ensemble).transpose(1, 0, 2))
    sb = jnp.stack([mk0[:choices, 0, :rows], mk1[:choices, 0, :rows]])
    samples = (sb.transpose(0, 2, 1)
               .reshape(_S, nnodes, ensemble, choices).transpose(0, 1, 3, 2))
    return samples, marginals
